# baseline (speedup 1.0000x reference)
import functools

import jax
import jax.numpy as jnp
from jax import lax
from jax.experimental import pallas as pl
from jax.experimental.pallas import tpu as pltpu
from jax.experimental.pallas import tpu_sc as plsc

N = 10000
E = 320000
D = 128

NC = 2
NS = 16
NW = NC * NS

K = 112
EPW = -(-E // NW)
EPW_PAD = -(-EPW // (2 * K)) * (2 * K)
CHUNKS = EPW_PAD // K
E_PAD = EPW_PAD * NW

ROWS_PER_TILE = 632
ACC_ROWS = NS * ROWS_PER_TILE


def _mm_body(h_ref, w_ref, norm_ref, o_ref):
    o_ref[...] = (
        jnp.dot(h_ref[...], w_ref[...], preferred_element_type=jnp.float32)
        * norm_ref[...]
    )


def _epilogue_body(p_ref, norm_ref, bias_ref, t_ref, o_ref):
    s = p_ref[0, :N] + p_ref[1, :N]
    o_ref[...] = jnp.maximum(s * norm_ref[...] * t_ref[0, 0] + bias_ref[...], 0.0)


def _scatter_body(hw, srcr, dstr, zeros, out,
                  src_v, dst_v, rows_v, acc, gsem):
    cid = lax.axis_index("c")
    sid = lax.axis_index("s")
    wid = cid * NS + sid

    r0 = sid * ROWS_PER_TILE
    pltpu.sync_copy(zeros, acc.at[pl.ds(r0, ROWS_PER_TILE)])

    pltpu.sync_copy(srcr.at[wid], src_v)
    pltpu.sync_copy(dstr.at[wid], dst_v)
    plsc.subcore_barrier()

    def gather_start(j, b):
        off = pl.multiple_of(j * K, 8)
        pltpu.async_copy(hw.at[src_v.at[pl.ds(off, K)]], rows_v.at[b],
                         gsem.at[b])

    def gather_wait(j, b):
        off = pl.multiple_of(j * K, 8)
        pltpu.make_async_copy(hw.at[src_v.at[pl.ds(off, K)]], rows_v.at[b],
                              gsem.at[b]).wait()

    gather_start(0, 0)
    gather_start(1, 1)

    def _step(it, carry):
        j = it * 2
        for b in range(2):
            gather_wait(j + b, b)
            offd = pl.multiple_of((j + b) * K, 8)
            pltpu.sync_copy(rows_v.at[b], acc.at[dst_v.at[pl.ds(offd, K)]],
                            add=True)
            gather_start(j + 2 + b, b)
        return carry

    lax.fori_loop(0, (CHUNKS - 2) // 2, _step, 0)

    for b in range(2):
        j = CHUNKS - 2 + b
        gather_wait(j, b)
        offd = pl.multiple_of(j * K, 8)
        pltpu.sync_copy(rows_v.at[b], acc.at[dst_v.at[pl.ds(offd, K)]],
                        add=True)

    plsc.subcore_barrier()

    pltpu.sync_copy(acc.at[pl.ds(r0, ROWS_PER_TILE)],
                    out.at[cid, pl.ds(r0, ROWS_PER_TILE)])


@functools.partial(jax.jit, static_argnums=())
def _scatter_call(hw, srcr, dstr, zeros):
    mesh = plsc.VectorSubcoreMesh(
        core_axis_name="c", subcore_axis_name="s", num_cores=NC, num_subcores=NS
    )
    return pl.kernel(
        _scatter_body,
        out_type=jax.ShapeDtypeStruct((NC, ACC_ROWS, D), jnp.float32),
        mesh=mesh,
        scratch_types=[
            pltpu.VMEM((EPW_PAD,), jnp.int32),
            pltpu.VMEM((EPW_PAD,), jnp.int32),
            pltpu.VMEM((2, K, D), jnp.float32),
            pltpu.VMEM_SHARED((ACC_ROWS, D), jnp.float32),
            pltpu.SemaphoreType.DMA((2,)),
        ],
    )(hw, srcr, dstr, zeros)


def kernel(t, h, edge_index, norm, weight, bias):
    hw = pl.pallas_call(
        _mm_body,
        out_shape=jax.ShapeDtypeStruct((N, D), jnp.float32),
    )(h, weight, norm)

    src = edge_index[0]
    dst = edge_index[1]
    pad = E_PAD - E
    srcr = jnp.pad(src, (0, pad)).reshape(NW, EPW_PAD)
    dstr = jnp.pad(dst, (0, pad), constant_values=N).reshape(NW, EPW_PAD)
    zeros = jnp.zeros((ROWS_PER_TILE, D), jnp.float32)

    parts = _scatter_call(hw, srcr, dstr, zeros)

    return pl.pallas_call(
        _epilogue_body,
        out_shape=jax.ShapeDtypeStruct((N, D), jnp.float32),
    )(parts, norm, bias, t.reshape(1, 1))


# speedup vs baseline: 1.0010x; 1.0010x over previous
import functools

import jax
import jax.numpy as jnp
from jax import lax
from jax.experimental import pallas as pl
from jax.experimental.pallas import tpu as pltpu
from jax.experimental.pallas import tpu_sc as plsc

N = 10000
E = 320000
D = 128

NC = 2
NS = 16
NW = NC * NS

K = 112
EPW = -(-E // NW)
EPW_PAD = -(-EPW // (2 * K)) * (2 * K)
CHUNKS = EPW_PAD // K
E_PAD = EPW_PAD * NW

ROWS_PER_TILE = 632
ACC_ROWS = NS * ROWS_PER_TILE


def _mm_body(h_ref, w_ref, norm_ref, o_ref):
    o_ref[...] = (
        jnp.dot(h_ref[...], w_ref[...], preferred_element_type=jnp.float32)
        * norm_ref[...]
    )


def _epilogue_body(p_ref, norm_ref, bias_ref, t_ref, o_ref):
    s = p_ref[0, :N] + p_ref[1, :N]
    o_ref[...] = jnp.maximum(s * norm_ref[...] * t_ref[0, 0] + bias_ref[...], 0.0)


def _scatter_body(hw, srcr, dstr, zeros, out,
                  src_v, dst_v, rows_v, acc, gsem):
    cid = lax.axis_index("c")
    sid = lax.axis_index("s")
    wid = cid * NS + sid

    r0 = sid * ROWS_PER_TILE
    pltpu.sync_copy(zeros, acc.at[pl.ds(r0, ROWS_PER_TILE)])

    pltpu.sync_copy(srcr.at[wid], src_v)
    pltpu.sync_copy(dstr.at[wid], dst_v)
    plsc.subcore_barrier()

    def gather_start(j, b):
        off = pl.multiple_of(j * K, 8)
        pltpu.async_copy(hw.at[src_v.at[pl.ds(off, K)]], rows_v.at[b],
                         gsem.at[b], priority=b)

    def gather_wait(j, b):
        off = pl.multiple_of(j * K, 8)
        pltpu.make_async_copy(hw.at[src_v.at[pl.ds(off, K)]], rows_v.at[b],
                              gsem.at[b]).wait()

    gather_start(0, 0)
    gather_start(1, 1)

    def _step(it, carry):
        j = it * 2
        for b in range(2):
            gather_wait(j + b, b)
            offd = pl.multiple_of((j + b) * K, 8)
            pltpu.sync_copy(rows_v.at[b], acc.at[dst_v.at[pl.ds(offd, K)]],
                            add=True)
            gather_start(j + 2 + b, b)
        return carry

    lax.fori_loop(0, (CHUNKS - 2) // 2, _step, 0)

    for b in range(2):
        j = CHUNKS - 2 + b
        gather_wait(j, b)
        offd = pl.multiple_of(j * K, 8)
        pltpu.sync_copy(rows_v.at[b], acc.at[dst_v.at[pl.ds(offd, K)]],
                        add=True)

    plsc.subcore_barrier()

    pltpu.sync_copy(acc.at[pl.ds(r0, ROWS_PER_TILE)],
                    out.at[cid, pl.ds(r0, ROWS_PER_TILE)])


@functools.partial(jax.jit, static_argnums=())
def _scatter_call(hw, srcr, dstr, zeros):
    mesh = plsc.VectorSubcoreMesh(
        core_axis_name="c", subcore_axis_name="s", num_cores=NC, num_subcores=NS
    )
    return pl.kernel(
        _scatter_body,
        out_type=jax.ShapeDtypeStruct((NC, ACC_ROWS, D), jnp.float32),
        mesh=mesh,
        scratch_types=[
            pltpu.VMEM((EPW_PAD,), jnp.int32),
            pltpu.VMEM((EPW_PAD,), jnp.int32),
            pltpu.VMEM((2, K, D), jnp.float32),
            pltpu.VMEM_SHARED((ACC_ROWS, D), jnp.float32),
            pltpu.SemaphoreType.DMA((2,)),
        ],
    )(hw, srcr, dstr, zeros)


def kernel(t, h, edge_index, norm, weight, bias):
    hw = pl.pallas_call(
        _mm_body,
        out_shape=jax.ShapeDtypeStruct((N, D), jnp.float32),
    )(h, weight, norm)

    src = edge_index[0]
    dst = edge_index[1]
    pad = E_PAD - E
    srcr = jnp.pad(src, (0, pad)).reshape(NW, EPW_PAD)
    dstr = jnp.pad(dst, (0, pad), constant_values=N).reshape(NW, EPW_PAD)
    zeros = jnp.zeros((ROWS_PER_TILE, D), jnp.float32)

    parts = _scatter_call(hw, srcr, dstr, zeros)

    return pl.pallas_call(
        _epilogue_body,
        out_shape=jax.ShapeDtypeStruct((N, D), jnp.float32),
    )(parts, norm, bias, t.reshape(1, 1))


# speedup vs baseline: 1.0017x; 1.0007x over previous
import functools

import jax
import jax.numpy as jnp
from jax import lax
from jax.experimental import pallas as pl
from jax.experimental.pallas import tpu as pltpu
from jax.experimental.pallas import tpu_sc as plsc

N = 10000
E = 320000
D = 128

NC = 2
NS = 16
NW = NC * NS

K = 112
EPW = -(-E // NW)
EPW_PAD = -(-EPW // (2 * K)) * (2 * K)
CHUNKS = EPW_PAD // K
E_PAD = EPW_PAD * NW

ROWS_PER_TILE = 632
ACC_ROWS = NS * ROWS_PER_TILE


def _mm_body(h_ref, w_ref, norm_ref, o_ref):
    o_ref[...] = (
        jnp.dot(h_ref[...], w_ref[...], preferred_element_type=jnp.float32)
        * norm_ref[...]
    )


def _epilogue_body(p_ref, norm_ref, bias_ref, t_ref, o_ref):
    s = p_ref[0, :N] + p_ref[1, :N]
    o_ref[...] = jnp.maximum(s * norm_ref[...] * t_ref[0, 0] + bias_ref[...], 0.0)


def _scatter_body(hw, srcr, dstr, zeros, out,
                  src_v, dst_v, rows_v, acc, gsem):
    cid = lax.axis_index("c")
    sid = lax.axis_index("s")
    wid = cid * NS + sid

    r0 = sid * ROWS_PER_TILE
    pltpu.sync_copy(zeros, acc.at[pl.ds(r0, ROWS_PER_TILE)])

    pltpu.sync_copy(srcr.at[wid], src_v)
    pltpu.sync_copy(dstr.at[wid], dst_v)
    plsc.subcore_barrier()

    def gather_start(j, b):
        off = pl.multiple_of(j * K, 8)
        pltpu.async_copy(hw.at[src_v.at[pl.ds(off, K)]], rows_v.at[b],
                         gsem.at[b])

    def gather_wait(j, b):
        off = pl.multiple_of(j * K, 8)
        pltpu.make_async_copy(hw.at[src_v.at[pl.ds(off, K)]], rows_v.at[b],
                              gsem.at[b]).wait()

    gather_start(0, 0)
    gather_start(1, 1)

    def _step(it, carry):
        j = it * 2
        for b in range(2):
            gather_wait(j + b, b)
            offd = pl.multiple_of((j + b) * K, 8)
            pltpu.sync_copy(rows_v.at[b], acc.at[dst_v.at[pl.ds(offd, K)]],
                            add=True)
            gather_start(j + 2 + b, b)
        return carry

    lax.fori_loop(0, (CHUNKS - 2) // 2, _step, 0)

    for b in range(2):
        j = CHUNKS - 2 + b
        gather_wait(j, b)
        offd = pl.multiple_of(j * K, 8)
        pltpu.sync_copy(rows_v.at[b], acc.at[dst_v.at[pl.ds(offd, K)]],
                        add=True)

    plsc.subcore_barrier()

    pltpu.sync_copy(acc.at[pl.ds(r0, ROWS_PER_TILE)],
                    out.at[cid, pl.ds(r0, ROWS_PER_TILE)])


@functools.partial(jax.jit, static_argnums=())
def _scatter_call(hw, srcr, dstr, zeros):
    mesh = plsc.VectorSubcoreMesh(
        core_axis_name="c", subcore_axis_name="s", num_cores=NC, num_subcores=NS
    )
    return pl.kernel(
        _scatter_body,
        out_type=jax.ShapeDtypeStruct((NC, ACC_ROWS, D), jnp.float32),
        mesh=mesh,
        scratch_types=[
            pltpu.VMEM((EPW_PAD,), jnp.int32),
            pltpu.VMEM((EPW_PAD,), jnp.int32),
            pltpu.VMEM((2, K, D), jnp.float32),
            pltpu.VMEM_SHARED((ACC_ROWS, D), jnp.float32),
            pltpu.SemaphoreType.DMA((2,)),
        ],
    )(hw, srcr, dstr, zeros)


def kernel(t, h, edge_index, norm, weight, bias):
    hw = pl.pallas_call(
        _mm_body,
        out_shape=jax.ShapeDtypeStruct((N, D), jnp.float32),
    )(h, weight, norm)

    src = edge_index[0]
    dst = edge_index[1]
    pad = E_PAD - E
    srcr = jnp.pad(src, (0, pad)).reshape(NW, EPW_PAD)
    dstr = jnp.pad(dst, (0, pad), constant_values=N).reshape(NW, EPW_PAD)
    zeros = jnp.zeros((ROWS_PER_TILE, D), jnp.float32)

    parts = _scatter_call(hw, srcr, dstr, zeros)

    return pl.pallas_call(
        _epilogue_body,
        out_shape=jax.ShapeDtypeStruct((N, D), jnp.float32),
    )(parts, norm, bias, t.reshape(1, 1))
